# baseline (speedup 1.0000x reference)
import functools

import jax
import jax.numpy as jnp
from jax import lax
from jax.experimental import pallas as pl
from jax.experimental.pallas import tpu as pltpu
from jax.experimental.pallas import tpu_sc as plsc
from jax._src.pallas import mpmd

_SCALE = 64.0
_MARGIN = 0.4
_MS = _SCALE * _MARGIN

_B = 1024
_C = 100000
_N = _B * _C

_ROWS_PER_BLOCK = 16

_NC = 2
_NS = 16
_NW = _NC * _NS
_LPW = _B // _NW
_L = 16


def _scale_body(cos_ref, out_ref):
    out_ref[...] = cos_ref[...] * _SCALE


_scale_call = pl.pallas_call(
    _scale_body,
    out_shape=jax.ShapeDtypeStruct((_B, _C), jnp.float32),
    grid=(_B // _ROWS_PER_BLOCK,),
    in_specs=[pl.BlockSpec((_ROWS_PER_BLOCK, _C), lambda i: (i, 0))],
    out_specs=pl.BlockSpec((_ROWS_PER_BLOCK, _C), lambda i: (i, 0)),
)


def _margin_body(src_hbm, label_hbm, out_hbm, lab_v, idx_v, val_v, sem):
    wid = lax.axis_index("s") * _NC + lax.axis_index("c")
    base = wid * _LPW
    pltpu.sync_copy(label_hbm.at[pl.ds(base, _LPW)], lab_v)
    for j in range(_LPW // _L):
        lab = lab_v[pl.ds(j * _L, _L)]
        row = base + j * _L + lax.iota(jnp.int32, _L)
        col = jnp.where(lab >= 0, lab, 0)
        idx_v[pl.ds(j * _L, _L)] = row * _C + col
    pltpu.async_copy(src_hbm.at[idx_v], val_v, sem).wait()
    for j in range(_LPW // _L):
        lab = lab_v[pl.ds(j * _L, _L)]
        v = val_v[pl.ds(j * _L, _L)]
        val_v[pl.ds(j * _L, _L)] = v - jnp.where(lab >= 0, _MS, 0.0)
    pltpu.async_copy(val_v, out_hbm.at[idx_v], sem).wait()


@functools.cache
def _margin_call():
    return mpmd._mpmd_map(
        [(plsc.VectorSubcoreMesh(core_axis_name="c", subcore_axis_name="s"),
          _margin_body)],
        out_types=jax.ShapeDtypeStruct((_N,), jnp.float32),
        input_output_aliases={0: 0},
        scratch_types=[
            pltpu.VMEM((_LPW,), jnp.int32),
            pltpu.VMEM((_LPW,), jnp.int32),
            pltpu.VMEM((_LPW,), jnp.float32),
            pltpu.SemaphoreType.DMA,
        ],
        name="cosface_margin_sc",
    )


def kernel(cosine, label):
    scaled = _scale_call(cosine)
    fixed = _margin_call()(scaled.reshape(_N), label.astype(jnp.int32))
    return fixed.reshape(_B, _C)


# speedup vs baseline: 2.1379x; 2.1379x over previous
import functools

import jax
import jax.numpy as jnp
from jax import lax
from jax.experimental import pallas as pl
from jax.experimental.pallas import tpu as pltpu
from jax.experimental.pallas import tpu_sc as plsc
from jax._src.pallas import mpmd

_SCALE = 64.0
_MARGIN = 0.4
_MS = _SCALE * _MARGIN

_B = 1024
_C = 100000
_N = _B * _C

_ROWS_PER_BLOCK = 16

_NC = 2
_NS = 16
_NW = _NC * _NS
_LPW = _B // _NW
_L = 16


def _scale_body(cos_ref, out_ref):
    out_ref[...] = cos_ref[...] * _SCALE


_scale_call = pl.pallas_call(
    _scale_body,
    out_shape=jax.ShapeDtypeStruct((_B, _C), jnp.float32),
    grid=(_B // _ROWS_PER_BLOCK,),
    in_specs=[pl.BlockSpec((_ROWS_PER_BLOCK, _C), lambda i: (i, 0))],
    out_specs=pl.BlockSpec((_ROWS_PER_BLOCK, _C), lambda i: (i, 0)),
)


def _fused_body(lab_ref, cos_ref, out_ref):
    cols = lax.broadcasted_iota(jnp.int32, (_ROWS_PER_BLOCK, _C), 1)
    hit = cols == lab_ref[...]
    out_ref[...] = cos_ref[...] * _SCALE - jnp.where(hit, _MS, 0.0)


_fused_call = pl.pallas_call(
    _fused_body,
    out_shape=jax.ShapeDtypeStruct((_B, _C), jnp.float32),
    grid=(_B // _ROWS_PER_BLOCK,),
    in_specs=[
        pl.BlockSpec((_ROWS_PER_BLOCK, 1), lambda i: (i, 0)),
        pl.BlockSpec((_ROWS_PER_BLOCK, _C), lambda i: (i, 0)),
    ],
    out_specs=pl.BlockSpec((_ROWS_PER_BLOCK, _C), lambda i: (i, 0)),
)


def _margin_body(src_hbm, label_hbm, out_hbm, lab_v, idx_v, val_v, sem):
    wid = lax.axis_index("s") * _NC + lax.axis_index("c")
    base = wid * _LPW
    pltpu.sync_copy(label_hbm.at[pl.ds(base, _LPW)], lab_v)
    for j in range(_LPW // _L):
        lab = lab_v[pl.ds(j * _L, _L)]
        row = base + j * _L + lax.iota(jnp.int32, _L)
        col = jnp.where(lab >= 0, lab, 0)
        idx_v[pl.ds(j * _L, _L)] = row * _C + col
    pltpu.async_copy(src_hbm.at[idx_v], val_v, sem).wait()
    for j in range(_LPW // _L):
        lab = lab_v[pl.ds(j * _L, _L)]
        v = val_v[pl.ds(j * _L, _L)]
        val_v[pl.ds(j * _L, _L)] = v - jnp.where(lab >= 0, _MS, 0.0)
    pltpu.async_copy(val_v, out_hbm.at[idx_v], sem).wait()


@functools.cache
def _margin_call():
    return mpmd._mpmd_map(
        [(plsc.VectorSubcoreMesh(core_axis_name="c", subcore_axis_name="s"),
          _margin_body)],
        out_types=jax.ShapeDtypeStruct((_N,), jnp.float32),
        input_output_aliases={0: 0},
        scratch_types=[
            pltpu.VMEM((_LPW,), jnp.int32),
            pltpu.VMEM((_LPW,), jnp.int32),
            pltpu.VMEM((_LPW,), jnp.float32),
            pltpu.SemaphoreType.DMA,
        ],
        name="cosface_margin_sc",
    )


def kernel(cosine, label):
    return _fused_call(label.astype(jnp.int32).reshape(_B, 1), cosine)
